# baseline (speedup 1.0000x reference)
import jax
import jax.numpy as jnp
from jax import lax
from jax.experimental import pallas as pl
from jax.experimental.pallas import tpu as pltpu
from jax.experimental.pallas import tpu_sc as plsc

VOCAB = 49408
D = 512
SEQ = 77
SEQ_PAD = 80
STYLE_POS = 2
N_CLS = 345
N_STYLE = 8
NC, NS = 2, 16
NW = NC * NS
TOKMAX = 11
CHUNKS = ((0, 8), (8, 24), (32, 24), (56, 24))


def _make_gather_body(r0, rlen):
    def _gather_body(tokens_hbm, table_hbm, comp_hbm, tok_all, buf_a, buf_b,
                     gsem_a, gsem_b, wsem_a, wsem_b):
        wid = lax.axis_index("s") * NC + lax.axis_index("c")
        c0 = wid * N_CLS // NW
        c1 = (wid + 1) * N_CLS // NW
        n = c1 - c0
        base = jnp.minimum(c0, N_CLS - TOKMAX)
        pltpu.sync_copy(
            tokens_hbm.at[pl.ds(pl.multiple_of(base * SEQ_PAD, 8), TOKMAX * SEQ_PAD)],
            tok_all,
        )
        k0 = c0 - base

        def idx_ref(i):
            return tok_all.at[
                pl.ds(pl.multiple_of((k0 + i) * SEQ_PAD + r0, 8), rlen)
            ]

        def per_pair(p, _):
            ia = 2 * p
            ib = ia + 1
            @pl.when(p > 0)
            def _():
                pltpu.make_async_copy(buf_a, comp_hbm.at[c0 + ia - 2], wsem_a).wait()
                pltpu.make_async_copy(buf_b, comp_hbm.at[c0 + ib - 2], wsem_b).wait()

            ga = pltpu.async_copy(table_hbm.at[idx_ref(ia)], buf_a, gsem_a)

            @pl.when(ib < n)
            def _():
                pltpu.async_copy(table_hbm.at[idx_ref(ib)], buf_b, gsem_b)

            ga.wait()
            pltpu.async_copy(buf_a, comp_hbm.at[c0 + ia], wsem_a)

            @pl.when(ib < n)
            def _():
                pltpu.make_async_copy(table_hbm.at[idx_ref(ib)], buf_b, gsem_b).wait()
                pltpu.async_copy(buf_b, comp_hbm.at[c0 + ib], wsem_b)

            return ()

        npairs = (n + 1) // 2
        lax.fori_loop(0, npairs, per_pair, ())
        pltpu.make_async_copy(buf_a, comp_hbm.at[c0 + 2 * npairs - 2], wsem_a).wait()

        @pl.when(2 * npairs - 1 < n)
        def _():
            pltpu.make_async_copy(
                buf_b, comp_hbm.at[c0 + 2 * npairs - 1], wsem_b
            ).wait()

    return _gather_body


def _make_bc_body(r0):
    def _bc_body(comp_ref, style_ref, out_ref):
        for j in range(8):

            @pl.when(pl.program_id(0) % 8 == j)
            def _(j=j):
                col = comp_ref[:, j, :]
                for s in range(N_STYLE):
                    out_ref[0, pl.ds(s * N_CLS, N_CLS), :] = col

        if r0 <= STYLE_POS < r0 + 8:

            @pl.when(r0 + pl.program_id(0) == STYLE_POS)
            def _():
                for s in range(N_STYLE):
                    out_ref[0, pl.ds(s * N_CLS, N_CLS), :] = jnp.broadcast_to(
                        style_ref[s][None, :], (N_CLS, D)
                    )

    return _bc_body


def kernel(tokens, token_table, style_embedding):
    tokens_flat = jnp.pad(tokens, ((0, 0), (0, SEQ_PAD - SEQ))).reshape(-1)
    styles = style_embedding.reshape(N_STYLE, D)

    compacts = []
    for k, (r0, rlen) in enumerate(CHUNKS):
        gather = pl.kernel(
            _make_gather_body(r0, rlen),
            out_type=jax.ShapeDtypeStruct((N_CLS, rlen, D), jnp.float32),
            mesh=plsc.VectorSubcoreMesh(
                core_axis_name="c", subcore_axis_name="s",
                num_cores=NC, num_subcores=NS,
            ),
            scratch_types=[
                pltpu.VMEM((TOKMAX * SEQ_PAD,), jnp.int32),
                pltpu.VMEM((rlen, D), jnp.float32),
                pltpu.VMEM((rlen, D), jnp.float32),
                pltpu.SemaphoreType.DMA,
                pltpu.SemaphoreType.DMA,
                pltpu.SemaphoreType.DMA,
                pltpu.SemaphoreType.DMA,
            ],
            name=f"sc_gather_{k}",
        )
        compacts.append(gather(tokens_flat, token_table))

    out_t = None
    for k, (r0, rlen) in enumerate(CHUNKS):
        rout = min(r0 + rlen, SEQ) - r0
        args = [compacts[k], styles]
        in_specs = [
            pl.BlockSpec((N_CLS, 8, D), lambda r: (0, r // 8, 0)),
            pl.BlockSpec((N_STYLE, D), lambda r: (0, 0)),
        ]
        io_alias = {}
        if out_t is not None:
            args.append(out_t)
            in_specs.append(pl.BlockSpec(memory_space=pltpu.MemorySpace.HBM))
            io_alias = {2: 0}

        def body(*refs, _r0=r0):
            _make_bc_body(_r0)(refs[0], refs[1], refs[-1])

        out_t = pl.pallas_call(
            body,
            grid=(rout,),
            in_specs=in_specs,
            out_specs=pl.BlockSpec(
                (1, N_STYLE * N_CLS, D), lambda r, _r0=r0: (_r0 + r, 0, 0)
            ),
            out_shape=jax.ShapeDtypeStruct((SEQ, N_STYLE * N_CLS, D), jnp.float32),
            input_output_aliases=io_alias,
            name=f"tc_broadcast_{k}",
        )(*args)
    return jnp.transpose(out_t, (1, 0, 2))


# speedup vs baseline: 1.0205x; 1.0205x over previous
import jax
import jax.numpy as jnp
from jax import lax
from jax.experimental import pallas as pl
from jax.experimental.pallas import tpu as pltpu
from jax.experimental.pallas import tpu_sc as plsc

VOCAB = 49408
D = 512
SEQ = 77
SEQ_PAD = 80
STYLE_POS = 2
N_CLS = 345
N_STYLE = 8
NC, NS = 2, 16
NW = NC * NS
TOKMAX = 11
CHUNKS = ((0, 16), (16, 16), (32, 16), (48, 16), (64, 16))


def _make_gather_body(r0, rlen):
    def _gather_body(tokens_hbm, table_hbm, comp_hbm, tok_all, buf_a, buf_b,
                     gsem_a, gsem_b, wsem_a, wsem_b):
        wid = lax.axis_index("s") * NC + lax.axis_index("c")
        c0 = wid * N_CLS // NW
        c1 = (wid + 1) * N_CLS // NW
        n = c1 - c0
        base = jnp.minimum(c0, N_CLS - TOKMAX)
        pltpu.sync_copy(
            tokens_hbm.at[pl.ds(pl.multiple_of(base * SEQ_PAD, 8), TOKMAX * SEQ_PAD)],
            tok_all,
        )
        k0 = c0 - base

        def idx_ref(i):
            return tok_all.at[
                pl.ds(pl.multiple_of((k0 + i) * SEQ_PAD + r0, 8), rlen)
            ]

        def per_pair(p, _):
            ia = 2 * p
            ib = ia + 1
            @pl.when(p > 0)
            def _():
                pltpu.make_async_copy(buf_a, comp_hbm.at[c0 + ia - 2], wsem_a).wait()
                pltpu.make_async_copy(buf_b, comp_hbm.at[c0 + ib - 2], wsem_b).wait()

            ga = pltpu.async_copy(table_hbm.at[idx_ref(ia)], buf_a, gsem_a)

            @pl.when(ib < n)
            def _():
                pltpu.async_copy(table_hbm.at[idx_ref(ib)], buf_b, gsem_b)

            ga.wait()
            pltpu.async_copy(buf_a, comp_hbm.at[c0 + ia], wsem_a)

            @pl.when(ib < n)
            def _():
                pltpu.make_async_copy(table_hbm.at[idx_ref(ib)], buf_b, gsem_b).wait()
                pltpu.async_copy(buf_b, comp_hbm.at[c0 + ib], wsem_b)

            return ()

        npairs = (n + 1) // 2
        lax.fori_loop(0, npairs, per_pair, ())
        pltpu.make_async_copy(buf_a, comp_hbm.at[c0 + 2 * npairs - 2], wsem_a).wait()

        @pl.when(2 * npairs - 1 < n)
        def _():
            pltpu.make_async_copy(
                buf_b, comp_hbm.at[c0 + 2 * npairs - 1], wsem_b
            ).wait()

    return _gather_body


def _make_bc_body(r0):
    def _bc_body(comp_ref, style_ref, out_ref):
        for j in range(8):

            @pl.when(pl.program_id(0) % 8 == j)
            def _(j=j):
                col = comp_ref[:, j, :]
                for s in range(N_STYLE):
                    out_ref[0, pl.ds(s * N_CLS, N_CLS), :] = col

        if r0 <= STYLE_POS < r0 + 8:

            @pl.when(r0 + pl.program_id(0) == STYLE_POS)
            def _():
                for s in range(N_STYLE):
                    out_ref[0, pl.ds(s * N_CLS, N_CLS), :] = jnp.broadcast_to(
                        style_ref[s][None, :], (N_CLS, D)
                    )

    return _bc_body


def kernel(tokens, token_table, style_embedding):
    tokens_flat = jnp.pad(tokens, ((0, 0), (0, SEQ_PAD - SEQ))).reshape(-1)
    styles = style_embedding.reshape(N_STYLE, D)

    compacts = []
    for k, (r0, rlen) in enumerate(CHUNKS):
        gather = pl.kernel(
            _make_gather_body(r0, rlen),
            out_type=jax.ShapeDtypeStruct((N_CLS, rlen, D), jnp.float32),
            mesh=plsc.VectorSubcoreMesh(
                core_axis_name="c", subcore_axis_name="s",
                num_cores=NC, num_subcores=NS,
            ),
            scratch_types=[
                pltpu.VMEM((TOKMAX * SEQ_PAD,), jnp.int32),
                pltpu.VMEM((rlen, D), jnp.float32),
                pltpu.VMEM((rlen, D), jnp.float32),
                pltpu.SemaphoreType.DMA,
                pltpu.SemaphoreType.DMA,
                pltpu.SemaphoreType.DMA,
                pltpu.SemaphoreType.DMA,
            ],
            name=f"sc_gather_{k}",
        )
        compacts.append(gather(tokens_flat, token_table))

    out_t = None
    for k, (r0, rlen) in enumerate(CHUNKS):
        rout = min(r0 + rlen, SEQ) - r0
        args = [compacts[k], styles]
        in_specs = [
            pl.BlockSpec((N_CLS, 8, D), lambda r: (0, r // 8, 0)),
            pl.BlockSpec((N_STYLE, D), lambda r: (0, 0)),
        ]
        io_alias = {}
        if out_t is not None:
            args.append(out_t)
            in_specs.append(pl.BlockSpec(memory_space=pltpu.MemorySpace.HBM))
            io_alias = {2: 0}

        def body(*refs, _r0=r0):
            _make_bc_body(_r0)(refs[0], refs[1], refs[-1])

        out_t = pl.pallas_call(
            body,
            grid=(rout,),
            in_specs=in_specs,
            out_specs=pl.BlockSpec(
                (1, N_STYLE * N_CLS, D), lambda r, _r0=r0: (_r0 + r, 0, 0)
            ),
            out_shape=jax.ShapeDtypeStruct((SEQ, N_STYLE * N_CLS, D), jnp.float32),
            input_output_aliases=io_alias,
            name=f"tc_broadcast_{k}",
        )(*args)
    return jnp.transpose(out_t, (1, 0, 2))
